# baseline (speedup 1.0000x reference)
import dataclasses
import functools

import jax
import jax.numpy as jnp
from jax import lax
from jax.experimental import pallas as pl
from jax.experimental.pallas import tpu as pltpu
from jax.experimental.pallas import tpu_sc as plsc

F32 = jnp.float32
NEG = 0.2
EPS = 1e-16
CLAMP = 50.0

NW = 32
NSUB = 16
TN = 1000

CB = 40
NSETS = 8


def _sc_compiler_params():
    cp = pltpu.CompilerParams()
    if "needs_layout_passes" in pltpu.CompilerParams.__dataclass_fields__:
        cp = dataclasses.replace(cp, needs_layout_passes=False)
    if "use_tc_tiling_on_sc" in pltpu.CompilerParams.__dataclass_fields__:
        cp = dataclasses.replace(cp, use_tc_tiling_on_sc=False)
    return cp


def _tc1_body(x_ref, wl_ref, bl_ref, wr_ref, br_ref, *out_refs):
    xb = x_ref[...]
    yl = jnp.dot(xb, wl_ref[...], preferred_element_type=F32,
                 precision=lax.Precision.HIGHEST) + bl_ref[...]
    yr = jnp.dot(xb, wr_ref[...], preferred_element_type=F32,
                 precision=lax.Precision.HIGHEST) + br_ref[...]
    for p in range(8):
        out_refs[0][p] = yl[:, p * 64:(p + 1) * 64]
        out_refs[1][p] = yr[:, p * 64:(p + 1) * 64]


def _tc1(x, W1l, b1l, W1r, b1r, N):
    nblk = N // TN
    full = lambda shape: pl.BlockSpec(shape, lambda i: (0, 0))
    return pl.pallas_call(
        _tc1_body,
        grid=(nblk,),
        in_specs=[
            pl.BlockSpec((TN, 128), lambda i: (i, 0)),
            full((128, 512)), full((1, 512)), full((128, 512)), full((1, 512)),
        ],
        out_specs=[pl.BlockSpec((8, TN, 64), lambda i: (0, i, 0))] * 2,
        out_shape=[jax.ShapeDtypeStruct((8, N, 64), F32)] * 2,
    )(x, W1l, b1l, W1r, b1r)


def _tc2_body(*refs):
    u_ref = refs[0]
    bias1_ref, w2l_ref, b2l_ref, w2r_ref, b2r_ref = refs[1:6]
    hl_ref, hr_ref = refs[6:]
    parts = []
    for p in range(8):
        ub = u_ref[p, 0] + u_ref[p, 1]
        parts.append(ub[:, :64] / (ub[:, 64:65] + EPS))
    h = jnp.concatenate(parts, axis=1) + bias1_ref[...]
    h = jnp.where(h > 0, h, jnp.exp(jnp.minimum(h, 0.0)) - 1.0)
    hl_ref[...] = jnp.dot(h, w2l_ref[...], preferred_element_type=F32,
                          precision=lax.Precision.HIGHEST) + b2l_ref[...]
    hr_ref[...] = jnp.dot(h, w2r_ref[...], preferred_element_type=F32,
                          precision=lax.Precision.HIGHEST) + b2r_ref[...]


def _tc2(us, bias1, W2lp, b2lp, W2rp, b2rp, N):
    nblk = N // TN
    ublock = pl.BlockSpec((8, 2, TN, 80), lambda i: (0, 0, i, 0))
    full = lambda shape: pl.BlockSpec(shape, lambda i: (0, 0))
    return pl.pallas_call(
        _tc2_body,
        grid=(nblk,),
        in_specs=[ublock,
                  full((1, 512)), full((512, 48)), full((1, 48)),
                  full((512, 48)), full((1, 48))],
        out_specs=[pl.BlockSpec((TN, 48), lambda i: (i, 0))] * 2,
        out_shape=[jax.ShapeDtypeStruct((N, 48), F32)] * 2,
    )(us, bias1, W2lp, b2lp, W2rp, b2rp)


def _tc3_body(u_ref, bias2_ref, o_ref):
    ub = u_ref[0] + u_ref[1]
    o_ref[...] = ub[:, :40] / (ub[:, 48:49] + EPS) + bias2_ref[...]


def _tc3(u2, bias2, N):
    nblk = N // TN
    return pl.pallas_call(
        _tc3_body,
        grid=(nblk,),
        in_specs=[pl.BlockSpec((2, TN, 64), lambda i: (0, i, 0)),
                  pl.BlockSpec((1, 40), lambda i: (0, 0))],
        out_specs=pl.BlockSpec((TN, 40), lambda i: (i, 0)),
        out_shape=jax.ShapeDtypeStruct((N, 40), F32),
    )(u2, bias2)


def _sc1(xls, xrs, src, dst, att1f, N, E):
    ew = E // NW
    nch = ew // CB
    rows_per_sub = (N // NSUB) // 8 * 8
    tail = N - rows_per_sub * NSUB
    mesh = plsc.VectorSubcoreMesh(core_axis_name="c", subcore_axis_name="s")

    scratch = [
        pltpu.VMEM_SHARED((N, 80), F32),
        pltpu.VMEM((ew,), jnp.int32),
        pltpu.VMEM((ew,), jnp.int32),
    ]
    scratch += [pltpu.VMEM((CB, 64), F32) for _ in range(2 * NSETS)]
    scratch += [pltpu.VMEM((CB, 80), F32) for _ in range(2)]
    scratch += [pltpu.VMEM((8, 64), F32)]
    scratch += [pltpu.SemaphoreType.DMA] * (2 * NSETS + 2)

    @functools.partial(
        pl.kernel,
        out_type=jax.ShapeDtypeStruct((8, 2, N, 80), F32),
        mesh=mesh,
        scratch_types=scratch,
        compiler_params=_sc_compiler_params(),
    )
    def sc1(*args):
        xlf, xrf, srcr, dstr, attr = args[:5]
        uout = args[5]
        (uacc, srcv, dstv) = args[6:9]
        rl = args[9:9 + NSETS]
        rr = args[9 + NSETS:9 + 2 * NSETS]
        wv = args[9 + 2 * NSETS:11 + 2 * NSETS]
        attv = args[11 + 2 * NSETS]
        gls = args[12 + 2 * NSETS:12 + 3 * NSETS]
        grs = args[12 + 3 * NSETS:12 + 4 * NSETS]
        ssem = args[12 + 4 * NSETS:14 + 4 * NSETS]

        cid = lax.axis_index("c")
        sid = lax.axis_index("s")
        wid = cid * NSUB + sid
        ebase = wid * ew
        pltpu.sync_copy(attr, attv)
        pltpu.sync_copy(srcr.at[pl.ds(ebase, ew)], srcv)
        pltpu.sync_copy(dstr.at[pl.ds(ebase, ew)], dstv)

        lane = lax.iota(jnp.int32, 16)
        is0 = lane == 0

        nfull = rows_per_sub // CB
        zrem = rows_per_sub - nfull * CB

        def fire(base, f, b):
            pltpu.async_copy(
                xlf.at[pl.ds(base, N)].at[srcv.at[pl.ds(f * CB, CB)]], rl[b], gls[b])
            pltpu.async_copy(
                xrf.at[pl.ds(base, N)].at[dstv.at[pl.ds(f * CB, CB)]], rr[b], grs[b])

        def chunk_body(base, m, b, att_regs):
            off = m * CB
            wb = b % 2
            pltpu.make_async_copy(
                xlf.at[pl.ds(base, N)].at[srcv.at[pl.ds(off, CB)]], rl[b], gls[b]).wait()
            pltpu.make_async_copy(
                xrf.at[pl.ds(base, N)].at[dstv.at[pl.ds(off, CB)]], rr[b], grs[b]).wait()

            @pl.when(m >= 2)
            def _():
                pltpu.make_async_copy(
                    wv[wb], uacc.at[dstv.at[pl.ds(off, CB)]], ssem[wb]).wait()

            rlb, rrb, wvb = rl[b], rr[b], wv[wb]

            @plsc.parallel_loop(0, CB, unroll=2)
            def _(i):
                lrow = []
                prods = []
                for j in range(4):
                    tl = rlb[i, pl.ds(16 * j, 16)]
                    t = tl + rrb[i, pl.ds(16 * j, 16)]
                    t = jnp.where(t > 0, t, NEG * t)
                    prods.append(t * att_regs[j])
                    lrow.append(tl)
                acc = (prods[0] + prods[1]) + (prods[2] + prods[3])
                a = jnp.minimum(jnp.maximum(jnp.sum(acc), -CLAMP), CLAMP)
                ex = jnp.exp(jnp.broadcast_to(a, (16,)))
                for j in range(4):
                    wvb[i, pl.ds(16 * j, 16)] = ex * lrow[j]
                wvb[i, pl.ds(64, 16)] = jnp.where(is0, ex, jnp.zeros((16,), F32))

            pltpu.async_copy(wvb, uacc.at[dstv.at[pl.ds(off, CB)]], ssem[wb],
                             add=True)
            f = m + (NSETS - 1)
            fb = (b + NSETS - 1) % NSETS

            @pl.when(f < nch)
            def _():
                fire(base, f, fb)

        @pl.loop(0, 8)
        def _(pp):
            base = pp * N
            for f in range(NSETS - 1):
                fire(base, f, f)

            @pl.loop(0, CB)
            def _(i):
                for j in range(5):
                    wv[0][i, pl.ds(16 * j, 16)] = jnp.zeros((16,), F32)

            for j in range(nfull):
                pltpu.sync_copy(wv[0], uacc.at[pl.ds(sid * rows_per_sub + j * CB, CB)])
            if zrem:
                pltpu.sync_copy(wv[0].at[pl.ds(0, zrem)],
                                uacc.at[pl.ds(sid * rows_per_sub + nfull * CB, zrem)])

            @pl.when(sid == 0)
            def _():
                pltpu.sync_copy(wv[0].at[pl.ds(0, tail)],
                                uacc.at[pl.ds(rows_per_sub * NSUB, tail)])

            plsc.subcore_barrier()

            att_regs = [attv[pp, pl.ds(16 * j, 16)] for j in range(4)]

            nmain = (nch - 1) // NSETS * NSETS

            @pl.loop(0, nmain, step=NSETS)
            def _(mb):
                for b in range(NSETS):
                    chunk_body(base, mb + b, b, att_regs)

            for t in range(nmain, nch):
                chunk_body(base, t, t % NSETS, att_regs)

            pltpu.make_async_copy(
                wv[1], uacc.at[dstv.at[pl.ds(0, CB)]], ssem[1]).wait()
            pltpu.make_async_copy(
                wv[0], uacc.at[dstv.at[pl.ds(0, CB)]], ssem[0]).wait()

            plsc.subcore_barrier()
            pltpu.sync_copy(uacc.at[pl.ds(sid * rows_per_sub, rows_per_sub)],
                            uout.at[pp, cid, pl.ds(sid * rows_per_sub, rows_per_sub)])

            @pl.when(sid == 0)
            def _():
                pltpu.sync_copy(uacc.at[pl.ds(rows_per_sub * NSUB, tail)],
                                uout.at[pp, cid, pl.ds(rows_per_sub * NSUB, tail)])

    return sc1(xls, xrs, src, dst, att1f)


def _sc2(hl, hr, src, dst, att48, N, E):
    ew = E // NW
    nch = ew // CB
    rows_per_sub = (N // NSUB) // 8 * 8
    tail = N - rows_per_sub * NSUB
    mesh = plsc.VectorSubcoreMesh(core_axis_name="c", subcore_axis_name="s")

    scratch = [
        pltpu.VMEM_SHARED((N, 64), F32),
        pltpu.VMEM((ew,), jnp.int32),
        pltpu.VMEM((ew,), jnp.int32),
    ]
    scratch += [pltpu.VMEM((CB, 48), F32) for _ in range(2 * NSETS)]
    scratch += [pltpu.VMEM((CB, 64), F32) for _ in range(2)]
    scratch += [pltpu.VMEM((48,), F32)]
    scratch += [pltpu.SemaphoreType.DMA] * (2 * NSETS + 2)

    @functools.partial(
        pl.kernel,
        out_type=jax.ShapeDtypeStruct((2, N, 64), F32),
        mesh=mesh,
        scratch_types=scratch,
        compiler_params=_sc_compiler_params(),
    )
    def sc2(*args):
        hlr, hrr, srcr, dstr, attr, uout = args[:6]
        (uacc, srcv, dstv) = args[6:9]
        rl = args[9:9 + NSETS]
        rr = args[9 + NSETS:9 + 2 * NSETS]
        wv = args[9 + 2 * NSETS:11 + 2 * NSETS]
        attv = args[11 + 2 * NSETS]
        gls = args[12 + 2 * NSETS:12 + 3 * NSETS]
        grs = args[12 + 3 * NSETS:12 + 4 * NSETS]
        ssem = args[12 + 4 * NSETS:14 + 4 * NSETS]

        cid = lax.axis_index("c")
        sid = lax.axis_index("s")
        wid = cid * NSUB + sid
        ebase = wid * ew
        pltpu.sync_copy(attr, attv)
        pltpu.sync_copy(srcr.at[pl.ds(ebase, ew)], srcv)
        pltpu.sync_copy(dstr.at[pl.ds(ebase, ew)], dstv)

        lane = lax.iota(jnp.int32, 16)
        is0 = lane == 0

        def fire(f, b):
            pltpu.async_copy(hlr.at[srcv.at[pl.ds(f * CB, CB)]], rl[b], gls[b])
            pltpu.async_copy(hrr.at[dstv.at[pl.ds(f * CB, CB)]], rr[b], grs[b])

        att_regs = [attv[pl.ds(16 * j, 16)] for j in range(3)]

        def chunk_body(m, b):
            off = m * CB
            wb = b % 2
            pltpu.make_async_copy(
                hlr.at[srcv.at[pl.ds(off, CB)]], rl[b], gls[b]).wait()
            pltpu.make_async_copy(
                hrr.at[dstv.at[pl.ds(off, CB)]], rr[b], grs[b]).wait()

            @pl.when(m >= 2)
            def _():
                pltpu.make_async_copy(
                    wv[wb], uacc.at[dstv.at[pl.ds(off, CB)]], ssem[wb]).wait()

            rlb, rrb, wvb = rl[b], rr[b], wv[wb]

            @plsc.parallel_loop(0, CB, unroll=2)
            def _(i):
                lrow = []
                prods = []
                for j in range(3):
                    tl = rlb[i, pl.ds(16 * j, 16)]
                    t = tl + rrb[i, pl.ds(16 * j, 16)]
                    t = jnp.where(t > 0, t, NEG * t)
                    prods.append(t * att_regs[j])
                    lrow.append(tl)
                acc = (prods[0] + prods[1]) + prods[2]
                a = jnp.minimum(jnp.maximum(jnp.sum(acc), -CLAMP), CLAMP)
                ex = jnp.exp(jnp.broadcast_to(a, (16,)))
                for j in range(3):
                    wvb[i, pl.ds(16 * j, 16)] = ex * lrow[j]
                wvb[i, pl.ds(48, 16)] = jnp.where(is0, ex, jnp.zeros((16,), F32))

            pltpu.async_copy(wvb, uacc.at[dstv.at[pl.ds(off, CB)]], ssem[wb],
                             add=True)
            f = m + (NSETS - 1)
            fb = (b + NSETS - 1) % NSETS

            @pl.when(f < nch)
            def _():
                fire(f, fb)

        for f in range(NSETS - 1):
            fire(f, f)

        @pl.loop(0, CB)
        def _(i):
            for j in range(4):
                wv[0][i, pl.ds(16 * j, 16)] = jnp.zeros((16,), F32)

        nfull = rows_per_sub // CB
        zrem = rows_per_sub - nfull * CB
        for j in range(nfull):
            pltpu.sync_copy(wv[0], uacc.at[pl.ds(sid * rows_per_sub + j * CB, CB)])
        if zrem:
            pltpu.sync_copy(wv[0].at[pl.ds(0, zrem)],
                            uacc.at[pl.ds(sid * rows_per_sub + nfull * CB, zrem)])

        @pl.when(sid == 0)
        def _():
            pltpu.sync_copy(wv[0].at[pl.ds(0, tail)],
                            uacc.at[pl.ds(rows_per_sub * NSUB, tail)])

        plsc.subcore_barrier()

        nmain = (nch - 1) // NSETS * NSETS

        @pl.loop(0, nmain, step=NSETS)
        def _(mb):
            for b in range(NSETS):
                chunk_body(mb + b, b)

        for t in range(nmain, nch):
            chunk_body(t, t % NSETS)

        pltpu.make_async_copy(
            wv[1], uacc.at[dstv.at[pl.ds(0, CB)]], ssem[1]).wait()
        pltpu.make_async_copy(
            wv[0], uacc.at[dstv.at[pl.ds(0, CB)]], ssem[0]).wait()

        plsc.subcore_barrier()
        pltpu.sync_copy(uacc.at[pl.ds(sid * rows_per_sub, rows_per_sub)],
                        uout.at[cid, pl.ds(sid * rows_per_sub, rows_per_sub)])

        @pl.when(sid == 0)
        def _():
            pltpu.sync_copy(uacc.at[pl.ds(rows_per_sub * NSUB, tail)],
                            uout.at[cid, pl.ds(rows_per_sub * NSUB, tail)])

    return sc2(hl, hr, src, dst, att48)


def kernel(x, edge_index, W1l, b1l, W1r, b1r, att1, bias1,
           W2l, b2l, W2r, b2r, att2, bias2):
    N = x.shape[0]
    E = edge_index.shape[1]
    src = edge_index[0]
    dst = edge_index[1]

    xl8, xr8 = _tc1(x, W1l, b1l.reshape(1, -1), W1r, b1r.reshape(1, -1), N)
    xlf = xl8.reshape(8 * N, 64)
    xrf = xr8.reshape(8 * N, 64)

    us = _sc1(xlf, xrf, src, dst, att1.reshape(8, 64), N, E)

    W2lp = jnp.pad(W2l, ((0, 0), (0, 8)))
    W2rp = jnp.pad(W2r, ((0, 0), (0, 8)))
    b2lp = jnp.pad(b2l, (0, 8)).reshape(1, -1)
    b2rp = jnp.pad(b2r, (0, 8)).reshape(1, -1)
    hl, hr = _tc2(us, bias1.reshape(1, -1), W2lp, b2lp, W2rp, b2rp, N)

    att48 = jnp.pad(att2.reshape(-1), (0, 8))
    u2 = _sc2(hl, hr, src, dst, att48, N, E)

    return _tc3(u2, bias2.reshape(1, -1), N)


# speedup vs baseline: 1.0880x; 1.0880x over previous
import dataclasses
import functools

import jax
import jax.numpy as jnp
from jax import lax
from jax.experimental import pallas as pl
from jax.experimental.pallas import tpu as pltpu
from jax.experimental.pallas import tpu_sc as plsc

F32 = jnp.float32
NEG = 0.2
EPS = 1e-16
CLAMP = 50.0

NW = 32
NSUB = 16
TN = 1000

CB = 80
NSETS = 4


def _sc_compiler_params():
    cp = pltpu.CompilerParams()
    if "needs_layout_passes" in pltpu.CompilerParams.__dataclass_fields__:
        cp = dataclasses.replace(cp, needs_layout_passes=False)
    if "use_tc_tiling_on_sc" in pltpu.CompilerParams.__dataclass_fields__:
        cp = dataclasses.replace(cp, use_tc_tiling_on_sc=False)
    return cp


def _tc1_body(x_ref, wl_ref, bl_ref, wr_ref, br_ref, *out_refs):
    xb = x_ref[...]
    yl = jnp.dot(xb, wl_ref[...], preferred_element_type=F32,
                 precision=lax.Precision.HIGHEST) + bl_ref[...]
    yr = jnp.dot(xb, wr_ref[...], preferred_element_type=F32,
                 precision=lax.Precision.HIGHEST) + br_ref[...]
    for p in range(8):
        out_refs[0][p] = yl[:, p * 64:(p + 1) * 64]
        out_refs[1][p] = yr[:, p * 64:(p + 1) * 64]


def _tc1(x, W1l, b1l, W1r, b1r, N):
    nblk = N // TN
    full = lambda shape: pl.BlockSpec(shape, lambda i: (0, 0))
    return pl.pallas_call(
        _tc1_body,
        grid=(nblk,),
        in_specs=[
            pl.BlockSpec((TN, 128), lambda i: (i, 0)),
            full((128, 512)), full((1, 512)), full((128, 512)), full((1, 512)),
        ],
        out_specs=[pl.BlockSpec((8, TN, 64), lambda i: (0, i, 0))] * 2,
        out_shape=[jax.ShapeDtypeStruct((8, N, 64), F32)] * 2,
    )(x, W1l, b1l, W1r, b1r)


def _tc2_body(*refs):
    u_ref = refs[0]
    bias1_ref, w2l_ref, b2l_ref, w2r_ref, b2r_ref = refs[1:6]
    hl_ref, hr_ref = refs[6:]
    parts = []
    for p in range(8):
        ub = u_ref[p, 0] + u_ref[p, 1]
        parts.append(ub[:, :64] / (ub[:, 64:65] + EPS))
    h = jnp.concatenate(parts, axis=1) + bias1_ref[...]
    h = jnp.where(h > 0, h, jnp.exp(jnp.minimum(h, 0.0)) - 1.0)
    hl_ref[...] = jnp.dot(h, w2l_ref[...], preferred_element_type=F32,
                          precision=lax.Precision.HIGHEST) + b2l_ref[...]
    hr_ref[...] = jnp.dot(h, w2r_ref[...], preferred_element_type=F32,
                          precision=lax.Precision.HIGHEST) + b2r_ref[...]


def _tc2(us, bias1, W2lp, b2lp, W2rp, b2rp, N):
    nblk = N // TN
    ublock = pl.BlockSpec((8, 2, TN, 80), lambda i: (0, 0, i, 0))
    full = lambda shape: pl.BlockSpec(shape, lambda i: (0, 0))
    return pl.pallas_call(
        _tc2_body,
        grid=(nblk,),
        in_specs=[ublock,
                  full((1, 512)), full((512, 48)), full((1, 48)),
                  full((512, 48)), full((1, 48))],
        out_specs=[pl.BlockSpec((TN, 48), lambda i: (i, 0))] * 2,
        out_shape=[jax.ShapeDtypeStruct((N, 48), F32)] * 2,
    )(us, bias1, W2lp, b2lp, W2rp, b2rp)


def _tc3_body(u_ref, bias2_ref, o_ref):
    ub = u_ref[0] + u_ref[1]
    o_ref[...] = ub[:, :40] / (ub[:, 48:49] + EPS) + bias2_ref[...]


def _tc3(u2, bias2, N):
    nblk = N // TN
    return pl.pallas_call(
        _tc3_body,
        grid=(nblk,),
        in_specs=[pl.BlockSpec((2, TN, 64), lambda i: (0, i, 0)),
                  pl.BlockSpec((1, 40), lambda i: (0, 0))],
        out_specs=pl.BlockSpec((TN, 40), lambda i: (i, 0)),
        out_shape=jax.ShapeDtypeStruct((N, 40), F32),
    )(u2, bias2)


def _sc1(xls, xrs, src, dst, att1f, N, E):
    ew = E // NW
    nch = ew // CB
    rows_per_sub = (N // NSUB) // 8 * 8
    tail = N - rows_per_sub * NSUB
    mesh = plsc.VectorSubcoreMesh(core_axis_name="c", subcore_axis_name="s")

    scratch = [
        pltpu.VMEM_SHARED((N, 80), F32),
        pltpu.VMEM((ew,), jnp.int32),
        pltpu.VMEM((ew,), jnp.int32),
    ]
    scratch += [pltpu.VMEM((CB, 64), F32) for _ in range(2 * NSETS)]
    scratch += [pltpu.VMEM((CB, 80), F32) for _ in range(2)]
    scratch += [pltpu.VMEM((8, 64), F32)]
    scratch += [pltpu.SemaphoreType.DMA] * (2 * NSETS + 2)

    @functools.partial(
        pl.kernel,
        out_type=jax.ShapeDtypeStruct((8, 2, N, 80), F32),
        mesh=mesh,
        scratch_types=scratch,
        compiler_params=_sc_compiler_params(),
    )
    def sc1(*args):
        xlf, xrf, srcr, dstr, attr = args[:5]
        uout = args[5]
        (uacc, srcv, dstv) = args[6:9]
        rl = args[9:9 + NSETS]
        rr = args[9 + NSETS:9 + 2 * NSETS]
        wv = args[9 + 2 * NSETS:11 + 2 * NSETS]
        attv = args[11 + 2 * NSETS]
        gls = args[12 + 2 * NSETS:12 + 3 * NSETS]
        grs = args[12 + 3 * NSETS:12 + 4 * NSETS]
        ssem = args[12 + 4 * NSETS:14 + 4 * NSETS]

        cid = lax.axis_index("c")
        sid = lax.axis_index("s")
        wid = cid * NSUB + sid
        ebase = wid * ew
        pltpu.sync_copy(attr, attv)
        pltpu.sync_copy(srcr.at[pl.ds(ebase, ew)], srcv)
        pltpu.sync_copy(dstr.at[pl.ds(ebase, ew)], dstv)

        lane = lax.iota(jnp.int32, 16)
        is0 = lane == 0

        nfull = rows_per_sub // CB
        zrem = rows_per_sub - nfull * CB

        def fire(base, f, b):
            pltpu.async_copy(
                xlf.at[pl.ds(base, N)].at[srcv.at[pl.ds(f * CB, CB)]], rl[b], gls[b])
            pltpu.async_copy(
                xrf.at[pl.ds(base, N)].at[dstv.at[pl.ds(f * CB, CB)]], rr[b], grs[b])

        def chunk_body(base, m, b, att_regs):
            off = m * CB
            wb = b % 2
            pltpu.make_async_copy(
                xlf.at[pl.ds(base, N)].at[srcv.at[pl.ds(off, CB)]], rl[b], gls[b]).wait()
            pltpu.make_async_copy(
                xrf.at[pl.ds(base, N)].at[dstv.at[pl.ds(off, CB)]], rr[b], grs[b]).wait()

            @pl.when(m >= 2)
            def _():
                pltpu.make_async_copy(
                    wv[wb], uacc.at[dstv.at[pl.ds(off, CB)]], ssem[wb]).wait()

            rlb, rrb, wvb = rl[b], rr[b], wv[wb]

            @plsc.parallel_loop(0, CB, unroll=2)
            def _(i):
                lrow = []
                prods = []
                for j in range(4):
                    tl = rlb[i, pl.ds(16 * j, 16)]
                    t = tl + rrb[i, pl.ds(16 * j, 16)]
                    t = jnp.where(t > 0, t, NEG * t)
                    prods.append(t * att_regs[j])
                    lrow.append(tl)
                acc = (prods[0] + prods[1]) + (prods[2] + prods[3])
                a = jnp.minimum(jnp.maximum(jnp.sum(acc), -CLAMP), CLAMP)
                ex = jnp.exp(jnp.broadcast_to(a, (16,)))
                for j in range(4):
                    wvb[i, pl.ds(16 * j, 16)] = ex * lrow[j]
                wvb[i, pl.ds(64, 16)] = jnp.where(is0, ex, jnp.zeros((16,), F32))

            pltpu.async_copy(wvb, uacc.at[dstv.at[pl.ds(off, CB)]], ssem[wb],
                             add=True)
            f = m + (NSETS - 1)
            fb = (b + NSETS - 1) % NSETS

            @pl.when(f < nch)
            def _():
                fire(base, f, fb)

        for f in range(NSETS - 1):
            fire(0, f, f)

        @pl.loop(0, 8)
        def _(pp):
            base = pp * N

            @pl.loop(0, CB)
            def _(i):
                for j in range(5):
                    wv[0][i, pl.ds(16 * j, 16)] = jnp.zeros((16,), F32)

            for j in range(nfull):
                pltpu.sync_copy(wv[0], uacc.at[pl.ds(sid * rows_per_sub + j * CB, CB)])
            if zrem:
                pltpu.sync_copy(wv[0].at[pl.ds(0, zrem)],
                                uacc.at[pl.ds(sid * rows_per_sub + nfull * CB, zrem)])

            @pl.when(sid == 0)
            def _():
                pltpu.sync_copy(wv[0].at[pl.ds(0, tail)],
                                uacc.at[pl.ds(rows_per_sub * NSUB, tail)])

            plsc.subcore_barrier()

            att_regs = [attv[pp, pl.ds(16 * j, 16)] for j in range(4)]

            nmain = (nch - 1) // NSETS * NSETS

            @pl.loop(0, nmain, step=NSETS)
            def _(mb):
                for b in range(NSETS):
                    chunk_body(base, mb + b, b, att_regs)

            for t in range(nmain, nch):
                chunk_body(base, t, t % NSETS, att_regs)

            @pl.when(pp < 7)
            def _():
                for f in range(NSETS - 1):
                    fire(base + N, f, f)

            pltpu.make_async_copy(
                wv[1], uacc.at[dstv.at[pl.ds(0, CB)]], ssem[1]).wait()
            pltpu.make_async_copy(
                wv[0], uacc.at[dstv.at[pl.ds(0, CB)]], ssem[0]).wait()

            plsc.subcore_barrier()
            pltpu.sync_copy(uacc.at[pl.ds(sid * rows_per_sub, rows_per_sub)],
                            uout.at[pp, cid, pl.ds(sid * rows_per_sub, rows_per_sub)])

            @pl.when(sid == 0)
            def _():
                pltpu.sync_copy(uacc.at[pl.ds(rows_per_sub * NSUB, tail)],
                                uout.at[pp, cid, pl.ds(rows_per_sub * NSUB, tail)])

    return sc1(xls, xrs, src, dst, att1f)


def _sc2(hl, hr, src, dst, att48, N, E):
    ew = E // NW
    nch = ew // CB
    rows_per_sub = (N // NSUB) // 8 * 8
    tail = N - rows_per_sub * NSUB
    mesh = plsc.VectorSubcoreMesh(core_axis_name="c", subcore_axis_name="s")

    scratch = [
        pltpu.VMEM_SHARED((N, 64), F32),
        pltpu.VMEM((ew,), jnp.int32),
        pltpu.VMEM((ew,), jnp.int32),
    ]
    scratch += [pltpu.VMEM((CB, 48), F32) for _ in range(2 * NSETS)]
    scratch += [pltpu.VMEM((CB, 64), F32) for _ in range(2)]
    scratch += [pltpu.VMEM((48,), F32)]
    scratch += [pltpu.SemaphoreType.DMA] * (2 * NSETS + 2)

    @functools.partial(
        pl.kernel,
        out_type=jax.ShapeDtypeStruct((2, N, 64), F32),
        mesh=mesh,
        scratch_types=scratch,
        compiler_params=_sc_compiler_params(),
    )
    def sc2(*args):
        hlr, hrr, srcr, dstr, attr, uout = args[:6]
        (uacc, srcv, dstv) = args[6:9]
        rl = args[9:9 + NSETS]
        rr = args[9 + NSETS:9 + 2 * NSETS]
        wv = args[9 + 2 * NSETS:11 + 2 * NSETS]
        attv = args[11 + 2 * NSETS]
        gls = args[12 + 2 * NSETS:12 + 3 * NSETS]
        grs = args[12 + 3 * NSETS:12 + 4 * NSETS]
        ssem = args[12 + 4 * NSETS:14 + 4 * NSETS]

        cid = lax.axis_index("c")
        sid = lax.axis_index("s")
        wid = cid * NSUB + sid
        ebase = wid * ew
        pltpu.sync_copy(attr, attv)
        pltpu.sync_copy(srcr.at[pl.ds(ebase, ew)], srcv)
        pltpu.sync_copy(dstr.at[pl.ds(ebase, ew)], dstv)

        lane = lax.iota(jnp.int32, 16)
        is0 = lane == 0

        def fire(f, b):
            pltpu.async_copy(hlr.at[srcv.at[pl.ds(f * CB, CB)]], rl[b], gls[b])
            pltpu.async_copy(hrr.at[dstv.at[pl.ds(f * CB, CB)]], rr[b], grs[b])

        att_regs = [attv[pl.ds(16 * j, 16)] for j in range(3)]

        def chunk_body(m, b):
            off = m * CB
            wb = b % 2
            pltpu.make_async_copy(
                hlr.at[srcv.at[pl.ds(off, CB)]], rl[b], gls[b]).wait()
            pltpu.make_async_copy(
                hrr.at[dstv.at[pl.ds(off, CB)]], rr[b], grs[b]).wait()

            @pl.when(m >= 2)
            def _():
                pltpu.make_async_copy(
                    wv[wb], uacc.at[dstv.at[pl.ds(off, CB)]], ssem[wb]).wait()

            rlb, rrb, wvb = rl[b], rr[b], wv[wb]

            @plsc.parallel_loop(0, CB, unroll=2)
            def _(i):
                lrow = []
                prods = []
                for j in range(3):
                    tl = rlb[i, pl.ds(16 * j, 16)]
                    t = tl + rrb[i, pl.ds(16 * j, 16)]
                    t = jnp.where(t > 0, t, NEG * t)
                    prods.append(t * att_regs[j])
                    lrow.append(tl)
                acc = (prods[0] + prods[1]) + prods[2]
                a = jnp.minimum(jnp.maximum(jnp.sum(acc), -CLAMP), CLAMP)
                ex = jnp.exp(jnp.broadcast_to(a, (16,)))
                for j in range(3):
                    wvb[i, pl.ds(16 * j, 16)] = ex * lrow[j]
                wvb[i, pl.ds(48, 16)] = jnp.where(is0, ex, jnp.zeros((16,), F32))

            pltpu.async_copy(wvb, uacc.at[dstv.at[pl.ds(off, CB)]], ssem[wb],
                             add=True)
            f = m + (NSETS - 1)
            fb = (b + NSETS - 1) % NSETS

            @pl.when(f < nch)
            def _():
                fire(f, fb)

        for f in range(NSETS - 1):
            fire(f, f)

        @pl.loop(0, CB)
        def _(i):
            for j in range(4):
                wv[0][i, pl.ds(16 * j, 16)] = jnp.zeros((16,), F32)

        nfull = rows_per_sub // CB
        zrem = rows_per_sub - nfull * CB
        for j in range(nfull):
            pltpu.sync_copy(wv[0], uacc.at[pl.ds(sid * rows_per_sub + j * CB, CB)])
        if zrem:
            pltpu.sync_copy(wv[0].at[pl.ds(0, zrem)],
                            uacc.at[pl.ds(sid * rows_per_sub + nfull * CB, zrem)])

        @pl.when(sid == 0)
        def _():
            pltpu.sync_copy(wv[0].at[pl.ds(0, tail)],
                            uacc.at[pl.ds(rows_per_sub * NSUB, tail)])

        plsc.subcore_barrier()

        nmain = (nch - 1) // NSETS * NSETS

        @pl.loop(0, nmain, step=NSETS)
        def _(mb):
            for b in range(NSETS):
                chunk_body(mb + b, b)

        for t in range(nmain, nch):
            chunk_body(t, t % NSETS)

        pltpu.make_async_copy(
            wv[1], uacc.at[dstv.at[pl.ds(0, CB)]], ssem[1]).wait()
        pltpu.make_async_copy(
            wv[0], uacc.at[dstv.at[pl.ds(0, CB)]], ssem[0]).wait()

        plsc.subcore_barrier()
        pltpu.sync_copy(uacc.at[pl.ds(sid * rows_per_sub, rows_per_sub)],
                        uout.at[cid, pl.ds(sid * rows_per_sub, rows_per_sub)])

        @pl.when(sid == 0)
        def _():
            pltpu.sync_copy(uacc.at[pl.ds(rows_per_sub * NSUB, tail)],
                            uout.at[cid, pl.ds(rows_per_sub * NSUB, tail)])

    return sc2(hl, hr, src, dst, att48)


def kernel(x, edge_index, W1l, b1l, W1r, b1r, att1, bias1,
           W2l, b2l, W2r, b2r, att2, bias2):
    N = x.shape[0]
    E = edge_index.shape[1]
    src = edge_index[0]
    dst = edge_index[1]

    xl8, xr8 = _tc1(x, W1l, b1l.reshape(1, -1), W1r, b1r.reshape(1, -1), N)
    xlf = xl8.reshape(8 * N, 64)
    xrf = xr8.reshape(8 * N, 64)

    us = _sc1(xlf, xrf, src, dst, att1.reshape(8, 64), N, E)

    W2lp = jnp.pad(W2l, ((0, 0), (0, 8)))
    W2rp = jnp.pad(W2r, ((0, 0), (0, 8)))
    b2lp = jnp.pad(b2l, (0, 8)).reshape(1, -1)
    b2rp = jnp.pad(b2r, (0, 8)).reshape(1, -1)
    hl, hr = _tc2(us, bias1.reshape(1, -1), W2lp, b2lp, W2rp, b2rp, N)

    att48 = jnp.pad(att2.reshape(-1), (0, 8))
    u2 = _sc2(hl, hr, src, dst, att48, N, E)

    return _tc3(u2, bias2.reshape(1, -1), N)
